# baseline (speedup 1.0000x reference)
import jax
import jax.numpy as jnp
from jax.experimental import pallas as pl
from jax.experimental.pallas import tpu as pltpu
from jax.experimental.pallas import tpu_sc as plsc

_NBUF = 2
_WORKERS = 32


def kernel(input_ids, table):
    batch, hist = input_ids.shape
    vocab, dim = table.shape
    rows_per_worker = batch // _WORKERS
    assert batch % _WORKERS == 0 and rows_per_worker % _NBUF == 0

    mesh = plsc.VectorSubcoreMesh(core_axis_name="c", subcore_axis_name="s")
    params = pltpu.CompilerParams(use_tc_tiling_on_sc=False)

    @jax.jit
    def run(table, idx):
        @pl.kernel(
            out_type=jax.ShapeDtypeStruct((batch, hist, dim), table.dtype),
            mesh=mesh,
            compiler_params=params,
            scratch_types=[
                pltpu.VMEM((rows_per_worker, hist), jnp.int32),
                pltpu.VMEM((_NBUF, hist, dim), table.dtype),
                pltpu.SemaphoreType.DMA((_NBUF,)),
                pltpu.SemaphoreType.DMA((_NBUF,)),
            ],
        )
        def gather_kernel(table_hbm, idx_hbm, out_hbm, idx_v, rows_v, gsem, wsem):
            wid = jax.lax.axis_index("s") * 2 + jax.lax.axis_index("c")
            wbase = wid * rows_per_worker
            pltpu.sync_copy(idx_hbm.at[pl.ds(wbase, rows_per_worker)], idx_v)

            @pl.loop(0, rows_per_worker, step=_NBUF)
            def _(g):
                gathers = []
                for b in range(_NBUF):
                    gathers.append(
                        pltpu.async_copy(
                            table_hbm.at[idx_v.at[g + b]], rows_v.at[b], gsem.at[b]
                        )
                    )
                writes = []
                for b in range(_NBUF):
                    gathers[b].wait()
                    writes.append(
                        pltpu.async_copy(
                            rows_v.at[b], out_hbm.at[wbase + g + b], wsem.at[b]
                        )
                    )
                for w in writes:
                    w.wait()

        return gather_kernel(table, idx)

    return run(table, input_ids)


# speedup vs baseline: 1.0021x; 1.0021x over previous
import jax
import jax.numpy as jnp
from jax.experimental import pallas as pl
from jax.experimental.pallas import tpu as pltpu
from jax.experimental.pallas import tpu_sc as plsc

_NBUF = 2
_WORKERS = 32


def kernel(input_ids, table):
    batch, hist = input_ids.shape
    vocab, dim = table.shape
    num_indices = batch * hist
    rows_per_worker = batch // _WORKERS
    assert batch % _WORKERS == 0 and rows_per_worker % _NBUF == 0

    mesh = plsc.VectorSubcoreMesh(core_axis_name="c", subcore_axis_name="s")
    params = pltpu.CompilerParams(use_tc_tiling_on_sc=False)

    @jax.jit
    def run(table, idx):
        @pl.kernel(
            out_type=jax.ShapeDtypeStruct((num_indices, dim), table.dtype),
            mesh=mesh,
            compiler_params=params,
            scratch_types=[
                pltpu.VMEM((rows_per_worker, hist), jnp.int32),
                pltpu.VMEM((_NBUF, hist, dim), table.dtype),
                pltpu.SemaphoreType.DMA((_NBUF,)),
                pltpu.SemaphoreType.DMA((_NBUF,)),
            ],
        )
        def gather_kernel(table_hbm, idx_hbm, out_hbm, idx_v, rows_v, gsem, wsem):
            wid = jax.lax.axis_index("s") * 2 + jax.lax.axis_index("c")
            wbase = wid * rows_per_worker
            pltpu.sync_copy(idx_hbm.at[pl.ds(wbase, rows_per_worker)], idx_v)

            @pl.loop(0, rows_per_worker, step=_NBUF)
            def _(g):
                gathers = []
                for b in range(_NBUF):
                    gathers.append(
                        pltpu.async_copy(
                            table_hbm.at[idx_v.at[g + b]], rows_v.at[b], gsem.at[b]
                        )
                    )
                writes = []
                for b in range(_NBUF):
                    gathers[b].wait()
                    writes.append(
                        pltpu.async_copy(
                            rows_v.at[b],
                            out_hbm.at[pl.ds((wbase + g + b) * hist, hist)],
                            wsem.at[b],
                        )
                    )
                for w in writes:
                    w.wait()

        out = gather_kernel(table, idx)
        return out.reshape(batch, hist, dim)

    return run(table, input_ids)


# speedup vs baseline: 1.2170x; 1.2145x over previous
import jax
import jax.numpy as jnp
from jax.experimental import pallas as pl
from jax.experimental.pallas import tpu as pltpu
from jax.experimental.pallas import tpu_sc as plsc

_NBUF = 2
_WORKERS = 32
_LANES = 128
_IDXPAD = 256


def kernel(input_ids, table):
    batch, hist = input_ids.shape
    vocab, dim = table.shape
    num_indices = batch * hist
    rows_per_worker = batch // _WORKERS
    assert batch % _WORKERS == 0 and rows_per_worker % _NBUF == 0

    mesh = plsc.VectorSubcoreMesh(core_axis_name="c", subcore_axis_name="s")

    @jax.jit
    def run(table, idx):
        padded = jnp.pad(table, ((0, 0), (0, _LANES - dim)))
        idxp = jnp.pad(idx, ((0, 0), (0, _IDXPAD - hist)), constant_values=-1)

        @pl.kernel(
            out_type=jax.ShapeDtypeStruct((num_indices, _LANES), table.dtype),
            mesh=mesh,
            scratch_types=[
                pltpu.VMEM((1, _IDXPAD), jnp.int32),
                pltpu.VMEM((1, _IDXPAD), jnp.int32),
                pltpu.VMEM((_NBUF, _IDXPAD, _LANES), table.dtype),
                pltpu.SemaphoreType.DMA((_NBUF,)),
                pltpu.SemaphoreType.DMA((_NBUF,)),
                pltpu.SemaphoreType.DMA((_NBUF,)),
            ],
        )
        def gather_kernel(
            table_hbm, idx_hbm, out_hbm, idx_a, idx_b, rows_v, isem, gsem, wsem
        ):
            wid = jax.lax.axis_index("s") * 2 + jax.lax.axis_index("c")
            rbase = wid * rows_per_worker
            idx_bufs = (idx_a, idx_b)

            def idx_dma(r, b):
                return pltpu.make_async_copy(
                    idx_hbm.at[pl.ds(rbase + r, 1)], idx_bufs[b], isem.at[b]
                )

            for b in range(_NBUF):
                idx_dma(b, b).start()

            @pl.loop(0, rows_per_worker, step=_NBUF)
            def _(g):
                gathers = []
                for b in range(_NBUF):
                    idx_dma(g + b, b).wait()
                    idx_row = plsc.Indices(idx_bufs[b].at[0], ignored_value=-1)
                    gathers.append(
                        pltpu.async_copy(
                            table_hbm.at[idx_row], rows_v.at[b], gsem.at[b]
                        )
                    )
                writes = []
                for b in range(_NBUF):
                    gathers[b].wait()

                    @pl.when(g + b + _NBUF < rows_per_worker)
                    def _():
                        idx_dma(g + b + _NBUF, b).start()

                    writes.append(
                        pltpu.async_copy(
                            rows_v.at[b].at[pl.ds(0, hist)],
                            out_hbm.at[pl.ds((rbase + g + b) * hist, hist)],
                            wsem.at[b],
                        )
                    )
                for w in writes:
                    w.wait()

        out = gather_kernel(padded, idxp)
        return out[:, :dim].reshape(batch, hist, dim)

    return run(table, input_ids)
